# Initial kernel scaffold; baseline (speedup 1.0000x reference)
import jax
import jax.numpy as jnp
from jax.experimental import pallas as pl


def kernel(c, pseudo_label):
    raise NotImplementedError("write your pallas kernel here")


# speedup vs baseline: 2.1497x; 2.1497x over previous
import jax
import jax.numpy as jnp
from jax.experimental import pallas as pl
from jax.experimental.pallas import tpu as pltpu

_R = 512


def _body(c_ref, y_ref, out_ref, counts_ref, s_ref):
    i = pl.program_id(0)

    @pl.when(i == 0)
    def _init():
        counts_ref[...] = jnp.zeros_like(counts_ref)
        s_ref[...] = jnp.zeros_like(s_ref)

    x = c_ref[...]
    y = y_ref[...]
    r, cnum = x.shape
    m = jnp.max(x, axis=1, keepdims=True)
    se = jnp.sum(jnp.exp(x - m), axis=1, keepdims=True)
    lse = jnp.log(se) + m
    col = jax.lax.broadcasted_iota(jnp.int32, (r, cnum), 1)
    onehot = col == y
    picked = jnp.sum(jnp.where(onehot, x, 0.0), axis=1, keepdims=True)
    nll = lse - picked
    counts_ref[...] += jnp.sum(onehot.astype(jnp.float32), axis=0, keepdims=True)
    s_ref[...] += jnp.sum(jnp.where(onehot, nll, 0.0), axis=0, keepdims=True)

    @pl.when(i == pl.num_programs(0) - 1)
    def _fini():
        counts = counts_ref[...]
        s = s_ref[...]
        pos = counts > 0.0
        k = jnp.sum(pos.astype(jnp.float32))
        ratio = jnp.where(pos, s / jnp.where(pos, counts, 1.0), 0.0)
        total = jnp.sum(ratio)
        loss = jnp.where(k > 0.0, total / jnp.where(k > 0.0, k, 1.0), 0.0)
        out_ref[...] = jnp.full((1, 1), loss, dtype=jnp.float32)


def kernel(c, pseudo_label):
    n, cnum = c.shape
    y2d = pseudo_label.reshape(n, 1).astype(jnp.int32)
    out = pl.pallas_call(
        _body,
        grid=(n // _R,),
        in_specs=[
            pl.BlockSpec((_R, cnum), lambda i: (i, 0)),
            pl.BlockSpec((_R, 1), lambda i: (i, 0)),
        ],
        out_specs=pl.BlockSpec((1, 1), lambda i: (0, 0)),
        out_shape=jax.ShapeDtypeStruct((1, 1), jnp.float32),
        scratch_shapes=[
            pltpu.VMEM((1, cnum), jnp.float32),
            pltpu.VMEM((1, cnum), jnp.float32),
        ],
    )(c, y2d)
    return out[0, 0]
